# baseline (speedup 1.0000x reference)
import jax
import jax.numpy as jnp
from jax.experimental import pallas as pl
from jax.experimental.pallas import tpu as pltpu

_UPPER = 20.0
_NEIGH8 = ((1, 0), (-1, 0), (0, -1), (0, 1), (-1, 1), (1, 1), (-1, -1), (1, -1))
_NEIGH9 = _NEIGH8 + ((0, 0),)


def _shift_edge(a, nx, ny):
    if nx == 1:
        a = jnp.concatenate([a[..., 1:, :], a[..., -1:, :]], axis=-2)
    elif nx == -1:
        a = jnp.concatenate([a[..., :1, :], a[..., :-1, :]], axis=-2)
    if ny == 1:
        a = jnp.concatenate([a[..., :, 1:], a[..., :, -1:]], axis=-1)
    elif ny == -1:
        a = jnp.concatenate([a[..., :, :1], a[..., :, :-1]], axis=-1)
    return a


def _stats_kernel(x_ref, t_ref, klm_ref, lse_ref, kls_ref, ce_ref):
    C, H, W = x_ref.shape[1], x_ref.shape[2], x_ref.shape[3]
    x = x_ref[0]
    t = t_ref[0, 0]
    m = jnp.max(x, axis=0)
    ex = jnp.exp(x - m[None])
    s = jnp.sum(ex, axis=0)
    L = x - m[None] - jnp.log(s)[None]
    P = ex * (1.0 / s)[None]
    E = jnp.sum(P * L, axis=0)

    ce = jnp.zeros((H, W), jnp.float32)
    for c in range(C):
        ce = ce + jnp.where(t == c, L[c], 0.0)
    ce_ref[0, 0] = jnp.full((1, 1), -jnp.sum(ce), jnp.float32)

    Pb = P.astype(jnp.bfloat16)
    Lb = L.astype(jnp.bfloat16)

    L_dn = jnp.concatenate([L[:, 1:, :], L[:, -1:, :]], axis=1)
    L_rt = jnp.concatenate([L[:, :, 1:], L[:, :, -1:]], axis=2)
    kls_ref[0, 0] = 2.0 * E - jnp.sum(P * L_dn, axis=0) - jnp.sum(P * L_rt, axis=0)

    kl_list = []
    for o, (nx, ny) in enumerate(_NEIGH8):
        acc = _shift_edge(E, nx, ny)
        for c in range(C):
            acc = acc - (_shift_edge(Pb[c], nx, ny) * Lb[c]).astype(jnp.float32)
        klm_ref[0, o] = acc.astype(jnp.bfloat16)
        kl_list.append(acc)
    m8 = kl_list[0]
    for ko in kl_list[1:]:
        m8 = jnp.maximum(m8, ko)
    se = jnp.zeros((H, W), jnp.float32)
    for ko in kl_list:
        se = se + jnp.exp(ko - m8)
    lse_ref[0, 0] = m8 + jnp.log(se)


def _dist_radius(gt):
    N, H, W = gt.shape
    INF = jnp.bfloat16(1e9)
    BIG = jnp.float32(1e5)

    dn = jnp.concatenate([gt[:, 1:, :], gt[:, -1:, :]], axis=1)
    rt = jnp.concatenate([gt[:, :, 1:], gt[:, :, -1:]], axis=2)
    bnd = jnp.logical_or(dn != gt, rt != gt)
    D0 = jnp.where(bnd, 0.0, 453.0)

    INF32 = jnp.float32(1e9)
    lane = jax.lax.broadcasted_iota(jnp.int32, (N, H, W), 2).astype(jnp.float32)
    u = D0 - lane
    v = D0 + lane
    for s in (1, 2, 4, 8, 16, 32, 64, 128):
        if s < W:
            u = jnp.minimum(u, jnp.concatenate(
                [jnp.full((N, H, s), INF32), u[:, :, : W - s]], axis=2))
            v = jnp.minimum(v, jnp.concatenate(
                [v[:, :, s:], jnp.full((N, H, s), INF32)], axis=2))
    D = jnp.minimum(D0, jnp.minimum(u + lane, v - lane)).astype(jnp.bfloat16)

    for s in range(8):
        J = 2 ** s
        t = D
        for sh in [2 ** k for k in range(s)] + [1]:
            t = jnp.minimum(t, jnp.concatenate(
                [jnp.full((N, H, sh), INF), t[:, :, : W - sh]], axis=2))
            t = jnp.minimum(t, jnp.concatenate(
                [t[:, :, sh:], jnp.full((N, H, sh), INF)], axis=2))
        up = jnp.concatenate([t[:, J:, :], jnp.full((N, J, W), INF)], axis=1)
        dd = jnp.concatenate([jnp.full((N, J, W), INF), t[:, : H - J, :]], axis=1)
        D = jnp.minimum(D, jnp.minimum(up, dd) + jnp.bfloat16(J))

    D32 = D.astype(jnp.float32)

    def shift_big(a, nx, ny):
        if nx == 1:
            a = jnp.concatenate([a[:, 1:, :], jnp.full((N, 1, W), BIG)], axis=1)
        elif nx == -1:
            a = jnp.concatenate([jnp.full((N, 1, W), BIG), a[:, :-1, :]], axis=1)
        if ny == 1:
            a = jnp.concatenate([a[:, :, 1:], jnp.full((N, H, 1), BIG)], axis=2)
        elif ny == -1:
            a = jnp.concatenate([jnp.full((N, H, 1), BIG), a[:, :, :-1]], axis=2)
        return a

    best = shift_big(D32, *_NEIGH9[0])
    bidx = jnp.zeros((N, H, W), jnp.int32)
    for k in range(1, 9):
        c = shift_big(D32, *_NEIGH9[k])
        take = c < best
        best = jnp.where(take, c, best)
        bidx = jnp.where(take, k, bidx)
    return bidx, jnp.minimum(D32, _UPPER) * (1.0 / _UPPER)


def _final_kernel(klm_ref, lse_ref, kls_ref, ce_ref, gt_ref, out_ref, eps_ref):
    N, _, H, W = kls_ref.shape
    pixel_ratio = jnp.float32(H * W * 0.05)

    rad, wgt = _dist_radius(gt_ref[:, 0])

    def build(k, e):
        eps_ref[k] = e
        return e * jnp.float32(1.2)

    jax.lax.fori_loop(0, 256, build, jnp.float32(1e-5))

    def count(e):
        return jnp.sum(jnp.where(kls_ref[...] > e, 1.0, 0.0))

    def bs(_, lohi):
        lo, hi = lohi
        mid = (lo + hi) // 2
        good = count(eps_ref[mid]) <= pixel_ratio
        return (jnp.where(good, lo, mid + 1), jnp.where(good, mid, hi))

    lo, _ = jax.lax.fori_loop(0, 8, bs, (jnp.int32(0), jnp.int32(255)))
    eps = eps_ref[lo]

    kb = jnp.where(kls_ref[...] > eps, 1.0, 0.0)[:, 0]

    def shift_zero(a, nx, ny):
        if nx == 1:
            a = jnp.concatenate([a[:, 1:, :], jnp.zeros((N, 1, W))], axis=1)
        elif nx == -1:
            a = jnp.concatenate([jnp.zeros((N, 1, W)), a[:, :-1, :]], axis=1)
        if ny == 1:
            a = jnp.concatenate([a[:, :, 1:], jnp.zeros((N, H, 1))], axis=2)
        elif ny == -1:
            a = jnp.concatenate([jnp.zeros((N, H, 1)), a[:, :, :-1]], axis=2)
        return a

    dil = kb
    for (nx, ny) in _NEIGH8:
        dil = jnp.maximum(dil, shift_zero(kb, nx, ny))

    keep = jnp.logical_and(dil > 0.0, rad != 8)

    pick = jnp.zeros((N, H, W), jnp.float32)
    for o in range(8):
        pick = pick + jnp.where(rad == o, klm_ref[:, o].astype(jnp.float32), 0.0)

    border = jnp.where(keep, lse_ref[:, 0] - pick + wgt, 0.0)
    total = jnp.sum(ce_ref[...]) + jnp.sum(border)
    out_ref[...] = jnp.full((1, 1), total, jnp.float32)


def kernel(slices, targets):
    N, C, H, W = slices.shape

    klm, lse, kls, ce = pl.pallas_call(
        _stats_kernel,
        grid=(N,),
        in_specs=[
            pl.BlockSpec((1, C, H, W), lambda n: (n, 0, 0, 0)),
            pl.BlockSpec((1, 1, H, W), lambda n: (n, 0, 0, 0)),
        ],
        out_specs=[
            pl.BlockSpec((1, 8, H, W), lambda n: (n, 0, 0, 0)),
            pl.BlockSpec((1, 1, H, W), lambda n: (n, 0, 0, 0)),
            pl.BlockSpec((1, 1, H, W), lambda n: (n, 0, 0, 0)),
            pl.BlockSpec((1, 1, 1, 1), lambda n: (n, 0, 0, 0)),
        ],
        out_shape=[
            jax.ShapeDtypeStruct((N, 8, H, W), jnp.bfloat16),
            jax.ShapeDtypeStruct((N, 1, H, W), jnp.float32),
            jax.ShapeDtypeStruct((N, 1, H, W), jnp.float32),
            jax.ShapeDtypeStruct((N, 1, 1, 1), jnp.float32),
        ],
    )(slices, targets)

    out = pl.pallas_call(
        _final_kernel,
        out_shape=jax.ShapeDtypeStruct((1, 1), jnp.float32),
        scratch_shapes=[pltpu.SMEM((256,), jnp.float32)],
    )(klm, lse, kls, ce, targets)
    return out[0, 0]


# speedup vs baseline: 1.0256x; 1.0256x over previous
import jax
import jax.numpy as jnp
from jax.experimental import pallas as pl
from jax.experimental.pallas import tpu as pltpu

_UPPER = 20.0
_NEIGH8 = ((1, 0), (-1, 0), (0, -1), (0, 1), (-1, 1), (1, 1), (-1, -1), (1, -1))
_NEIGH9 = _NEIGH8 + ((0, 0),)


def _shift_edge(a, nx, ny):
    if nx == 1:
        a = jnp.concatenate([a[..., 1:, :], a[..., -1:, :]], axis=-2)
    elif nx == -1:
        a = jnp.concatenate([a[..., :1, :], a[..., :-1, :]], axis=-2)
    if ny == 1:
        a = jnp.concatenate([a[..., :, 1:], a[..., :, -1:]], axis=-1)
    elif ny == -1:
        a = jnp.concatenate([a[..., :, :1], a[..., :, :-1]], axis=-1)
    return a


def _stats_kernel(x_ref, t_ref, klm_ref, lse_ref, kls_ref, ce_ref):
    C, H, W = x_ref.shape[1], x_ref.shape[2], x_ref.shape[3]
    x = x_ref[0]
    t = t_ref[0, 0]
    m = jnp.max(x, axis=0)
    ex = jnp.exp(x - m[None])
    s = jnp.sum(ex, axis=0)
    L = x - m[None] - jnp.log(s)[None]
    P = ex * (1.0 / s)[None]
    E = jnp.sum(P * L, axis=0)

    ce = jnp.zeros((H, W), jnp.float32)
    for c in range(C):
        ce = ce + jnp.where(t == c, L[c], 0.0)
    ce_ref[0, 0] = jnp.full((1, 1), -jnp.sum(ce), jnp.float32)

    Pb = P.astype(jnp.bfloat16)
    Lb = L.astype(jnp.bfloat16)

    L_dn = jnp.concatenate([L[:, 1:, :], L[:, -1:, :]], axis=1)
    L_rt = jnp.concatenate([L[:, :, 1:], L[:, :, -1:]], axis=2)
    kls_ref[0, 0] = 2.0 * E - jnp.sum(P * L_dn, axis=0) - jnp.sum(P * L_rt, axis=0)

    kl_list = []
    for o, (nx, ny) in enumerate(_NEIGH8):
        acc = _shift_edge(E, nx, ny)
        for c in range(C):
            acc = acc - (_shift_edge(Pb[c], nx, ny) * Lb[c]).astype(jnp.float32)
        klm_ref[0, o] = acc.astype(jnp.bfloat16)
        kl_list.append(acc)
    m8 = kl_list[0]
    for ko in kl_list[1:]:
        m8 = jnp.maximum(m8, ko)
    se = jnp.zeros((H, W), jnp.float32)
    for ko in kl_list:
        se = se + jnp.exp(ko - m8)
    lse_ref[0, 0] = m8 + jnp.log(se)


def _dist_radius(gt):
    N, H, W = gt.shape
    INF = jnp.bfloat16(1e9)
    BIG = jnp.float32(1e5)

    dn = jnp.concatenate([gt[:, 1:, :], gt[:, -1:, :]], axis=1)
    rt = jnp.concatenate([gt[:, :, 1:], gt[:, :, -1:]], axis=2)
    bnd = jnp.logical_or(dn != gt, rt != gt)
    D = jnp.where(bnd, 0.0, 453.0).astype(jnp.bfloat16)

    lane = jax.lax.broadcasted_iota(jnp.int32, (N, H, W), 2).astype(jnp.bfloat16)
    u = D - lane
    v = D + lane
    for s in (1, 2, 4, 8, 16, 32, 64, 128):
        if s < W:
            u = jnp.minimum(u, jnp.concatenate(
                [jnp.full((N, H, s), INF), u[:, :, : W - s]], axis=2))
            v = jnp.minimum(v, jnp.concatenate(
                [v[:, :, s:], jnp.full((N, H, s), INF)], axis=2))
    D = jnp.minimum(D, jnp.minimum(u + lane, v - lane))

    for s in range(8):
        J = 2 ** s
        t = D
        for sh in [2 ** k for k in range(s)] + [1]:
            t = jnp.minimum(t, jnp.concatenate(
                [jnp.full((N, H, sh), INF), t[:, :, : W - sh]], axis=2))
            t = jnp.minimum(t, jnp.concatenate(
                [t[:, :, sh:], jnp.full((N, H, sh), INF)], axis=2))
        up = jnp.concatenate([t[:, J:, :], jnp.full((N, J, W), INF)], axis=1)
        dd = jnp.concatenate([jnp.full((N, J, W), INF), t[:, : H - J, :]], axis=1)
        D = jnp.minimum(D, jnp.minimum(up, dd) + jnp.bfloat16(J))

    D32 = D.astype(jnp.float32)

    def shift_big(a, nx, ny):
        if nx == 1:
            a = jnp.concatenate([a[:, 1:, :], jnp.full((N, 1, W), BIG)], axis=1)
        elif nx == -1:
            a = jnp.concatenate([jnp.full((N, 1, W), BIG), a[:, :-1, :]], axis=1)
        if ny == 1:
            a = jnp.concatenate([a[:, :, 1:], jnp.full((N, H, 1), BIG)], axis=2)
        elif ny == -1:
            a = jnp.concatenate([jnp.full((N, H, 1), BIG), a[:, :, :-1]], axis=2)
        return a

    best = shift_big(D32, *_NEIGH9[0])
    bidx = jnp.zeros((N, H, W), jnp.int32)
    for k in range(1, 9):
        c = shift_big(D32, *_NEIGH9[k])
        take = c < best
        best = jnp.where(take, c, best)
        bidx = jnp.where(take, k, bidx)
    return bidx, jnp.minimum(D32, _UPPER) * (1.0 / _UPPER)


def _final_kernel(klm_ref, lse_ref, kls_ref, ce_ref, gt_ref, out_ref, eps_ref):
    N, _, H, W = kls_ref.shape
    pixel_ratio = jnp.float32(H * W * 0.05)

    rad, wgt = _dist_radius(gt_ref[:, 0])

    def build(k, e):
        eps_ref[k] = e
        return e * jnp.float32(1.2)

    jax.lax.fori_loop(0, 256, build, jnp.float32(1e-5))

    def count(e):
        return jnp.sum(jnp.where(kls_ref[...] > e, 1.0, 0.0))

    def bs(_, lohi):
        lo, hi = lohi
        mid = (lo + hi) // 2
        good = count(eps_ref[mid]) <= pixel_ratio
        return (jnp.where(good, lo, mid + 1), jnp.where(good, mid, hi))

    lo, _ = jax.lax.fori_loop(0, 8, bs, (jnp.int32(0), jnp.int32(255)))
    eps = eps_ref[lo]

    kb = jnp.where(kls_ref[...] > eps, 1.0, 0.0)[:, 0]

    def shift_zero(a, nx, ny):
        if nx == 1:
            a = jnp.concatenate([a[:, 1:, :], jnp.zeros((N, 1, W))], axis=1)
        elif nx == -1:
            a = jnp.concatenate([jnp.zeros((N, 1, W)), a[:, :-1, :]], axis=1)
        if ny == 1:
            a = jnp.concatenate([a[:, :, 1:], jnp.zeros((N, H, 1))], axis=2)
        elif ny == -1:
            a = jnp.concatenate([jnp.zeros((N, H, 1)), a[:, :, :-1]], axis=2)
        return a

    dil = kb
    for (nx, ny) in _NEIGH8:
        dil = jnp.maximum(dil, shift_zero(kb, nx, ny))

    keep = jnp.logical_and(dil > 0.0, rad != 8)

    pick = jnp.zeros((N, H, W), jnp.float32)
    for o in range(8):
        pick = pick + jnp.where(rad == o, klm_ref[:, o].astype(jnp.float32), 0.0)

    border = jnp.where(keep, lse_ref[:, 0] - pick + wgt, 0.0)
    total = jnp.sum(ce_ref[...]) + jnp.sum(border)
    out_ref[...] = jnp.full((1, 1), total, jnp.float32)


def kernel(slices, targets):
    N, C, H, W = slices.shape

    klm, lse, kls, ce = pl.pallas_call(
        _stats_kernel,
        grid=(N,),
        in_specs=[
            pl.BlockSpec((1, C, H, W), lambda n: (n, 0, 0, 0)),
            pl.BlockSpec((1, 1, H, W), lambda n: (n, 0, 0, 0)),
        ],
        out_specs=[
            pl.BlockSpec((1, 8, H, W), lambda n: (n, 0, 0, 0)),
            pl.BlockSpec((1, 1, H, W), lambda n: (n, 0, 0, 0)),
            pl.BlockSpec((1, 1, H, W), lambda n: (n, 0, 0, 0)),
            pl.BlockSpec((1, 1, 1, 1), lambda n: (n, 0, 0, 0)),
        ],
        out_shape=[
            jax.ShapeDtypeStruct((N, 8, H, W), jnp.bfloat16),
            jax.ShapeDtypeStruct((N, 1, H, W), jnp.float32),
            jax.ShapeDtypeStruct((N, 1, H, W), jnp.float32),
            jax.ShapeDtypeStruct((N, 1, 1, 1), jnp.float32),
        ],
    )(slices, targets)

    out = pl.pallas_call(
        _final_kernel,
        out_shape=jax.ShapeDtypeStruct((1, 1), jnp.float32),
        scratch_shapes=[pltpu.SMEM((256,), jnp.float32)],
    )(klm, lse, kls, ce, targets)
    return out[0, 0]


# speedup vs baseline: 1.0327x; 1.0069x over previous
import jax
import jax.numpy as jnp
from jax.experimental import pallas as pl
from jax.experimental.pallas import tpu as pltpu

_UPPER = 20.0
_NEIGH8 = ((1, 0), (-1, 0), (0, -1), (0, 1), (-1, 1), (1, 1), (-1, -1), (1, -1))
_NEIGH9 = _NEIGH8 + ((0, 0),)


def _shift_edge(a, nx, ny):
    if nx == 1:
        a = jnp.concatenate([a[..., 1:, :], a[..., -1:, :]], axis=-2)
    elif nx == -1:
        a = jnp.concatenate([a[..., :1, :], a[..., :-1, :]], axis=-2)
    if ny == 1:
        a = jnp.concatenate([a[..., :, 1:], a[..., :, -1:]], axis=-1)
    elif ny == -1:
        a = jnp.concatenate([a[..., :, :1], a[..., :, :-1]], axis=-1)
    return a


def _stats_kernel(x_ref, t_ref, klm_ref, lse_ref, kls_ref, ce_ref):
    C, H, W = x_ref.shape[1], x_ref.shape[2], x_ref.shape[3]
    x = x_ref[0]
    t = t_ref[0, 0]
    m = jnp.max(x, axis=0)
    ex = jnp.exp(x - m[None])
    s = jnp.sum(ex, axis=0)
    L = x - m[None] - jnp.log(s)[None]
    P = ex * (1.0 / s)[None]
    E = jnp.sum(P * L, axis=0)

    ce = jnp.zeros((H, W), jnp.float32)
    for c in range(C):
        ce = ce + jnp.where(t == c, L[c], 0.0)
    ce_ref[0, 0] = -ce

    Pb = P.astype(jnp.bfloat16)
    Lb = L.astype(jnp.bfloat16)

    L_dn = jnp.concatenate([L[:, 1:, :], L[:, -1:, :]], axis=1)
    L_rt = jnp.concatenate([L[:, :, 1:], L[:, :, -1:]], axis=2)
    kls_ref[0, 0] = 2.0 * E - jnp.sum(P * L_dn, axis=0) - jnp.sum(P * L_rt, axis=0)

    kl_list = []
    for o, (nx, ny) in enumerate(_NEIGH8):
        acc = _shift_edge(E, nx, ny)
        for c in range(C):
            acc = acc - (_shift_edge(Pb[c], nx, ny) * Lb[c]).astype(jnp.float32)
        klm_ref[0, o] = acc.astype(jnp.bfloat16)
        kl_list.append(acc)
    m8 = kl_list[0]
    for ko in kl_list[1:]:
        m8 = jnp.maximum(m8, ko)
    se = jnp.zeros((H, W), jnp.float32)
    for ko in kl_list:
        se = se + jnp.exp(ko - m8)
    lse_ref[0, 0] = m8 + jnp.log(se)


def _dist_radius(gt):
    N, H, W = gt.shape
    INF = jnp.bfloat16(1e9)
    BIG = jnp.float32(1e5)

    dn = jnp.concatenate([gt[:, 1:, :], gt[:, -1:, :]], axis=1)
    rt = jnp.concatenate([gt[:, :, 1:], gt[:, :, -1:]], axis=2)
    bnd = jnp.logical_or(dn != gt, rt != gt)
    D = jnp.where(bnd, 0.0, 453.0).astype(jnp.bfloat16)

    lane = jax.lax.broadcasted_iota(jnp.int32, (N, H, W), 2).astype(jnp.bfloat16)
    u = D - lane
    v = D + lane
    for s in (1, 2, 4, 8, 16, 32, 64, 128):
        if s < W:
            u = jnp.minimum(u, jnp.concatenate(
                [jnp.full((N, H, s), INF), u[:, :, : W - s]], axis=2))
            v = jnp.minimum(v, jnp.concatenate(
                [v[:, :, s:], jnp.full((N, H, s), INF)], axis=2))
    D = jnp.minimum(D, jnp.minimum(u + lane, v - lane))

    for s in range(8):
        J = 2 ** s
        t = D
        for sh in [2 ** k for k in range(s)] + [1]:
            t = jnp.minimum(t, jnp.concatenate(
                [jnp.full((N, H, sh), INF), t[:, :, : W - sh]], axis=2))
            t = jnp.minimum(t, jnp.concatenate(
                [t[:, :, sh:], jnp.full((N, H, sh), INF)], axis=2))
        up = jnp.concatenate([t[:, J:, :], jnp.full((N, J, W), INF)], axis=1)
        dd = jnp.concatenate([jnp.full((N, J, W), INF), t[:, : H - J, :]], axis=1)
        D = jnp.minimum(D, jnp.minimum(up, dd) + jnp.bfloat16(J))

    D32 = D.astype(jnp.float32)

    def shift_big(a, nx, ny):
        if nx == 1:
            a = jnp.concatenate([a[:, 1:, :], jnp.full((N, 1, W), BIG)], axis=1)
        elif nx == -1:
            a = jnp.concatenate([jnp.full((N, 1, W), BIG), a[:, :-1, :]], axis=1)
        if ny == 1:
            a = jnp.concatenate([a[:, :, 1:], jnp.full((N, H, 1), BIG)], axis=2)
        elif ny == -1:
            a = jnp.concatenate([jnp.full((N, H, 1), BIG), a[:, :, :-1]], axis=2)
        return a

    best = shift_big(D32, *_NEIGH9[0])
    bidx = jnp.zeros((N, H, W), jnp.int32)
    for k in range(1, 9):
        c = shift_big(D32, *_NEIGH9[k])
        take = c < best
        best = jnp.where(take, c, best)
        bidx = jnp.where(take, k, bidx)
    return bidx, jnp.minimum(D32, _UPPER) * (1.0 / _UPPER)


def _final_kernel(klm_ref, lse_ref, kls_ref, ce_ref, gt_ref, out_ref, eps_ref):
    N, _, H, W = kls_ref.shape
    pixel_ratio = jnp.float32(H * W * 0.05)

    rad, wgt = _dist_radius(gt_ref[:, 0])

    def build(k, e):
        eps_ref[k] = e
        return e * jnp.float32(1.2)

    jax.lax.fori_loop(0, 256, build, jnp.float32(1e-5))

    def count(e):
        return jnp.sum(jnp.where(kls_ref[...] > e, 1.0, 0.0))

    def bs(_, lohi):
        lo, hi = lohi
        mid = (lo + hi) // 2
        good = count(eps_ref[mid]) <= pixel_ratio
        return (jnp.where(good, lo, mid + 1), jnp.where(good, mid, hi))

    lo, _ = jax.lax.fori_loop(0, 8, bs, (jnp.int32(0), jnp.int32(255)))
    eps = eps_ref[lo]

    kb = jnp.where(kls_ref[...] > eps, 1.0, 0.0)[:, 0]

    def shift_zero(a, nx, ny):
        if nx == 1:
            a = jnp.concatenate([a[:, 1:, :], jnp.zeros((N, 1, W))], axis=1)
        elif nx == -1:
            a = jnp.concatenate([jnp.zeros((N, 1, W)), a[:, :-1, :]], axis=1)
        if ny == 1:
            a = jnp.concatenate([a[:, :, 1:], jnp.zeros((N, H, 1))], axis=2)
        elif ny == -1:
            a = jnp.concatenate([jnp.zeros((N, H, 1)), a[:, :, :-1]], axis=2)
        return a

    dil = kb
    for (nx, ny) in _NEIGH8:
        dil = jnp.maximum(dil, shift_zero(kb, nx, ny))

    keep = jnp.logical_and(dil > 0.0, rad != 8)

    pick = jnp.zeros((N, H, W), jnp.float32)
    for o in range(8):
        pick = pick + jnp.where(rad == o, klm_ref[:, o].astype(jnp.float32), 0.0)

    border = jnp.where(keep, lse_ref[:, 0] - pick + wgt, 0.0)
    total = jnp.sum(ce_ref[...]) + jnp.sum(border)
    out_ref[...] = jnp.full((1, 1), total, jnp.float32)


def kernel(slices, targets):
    N, C, H, W = slices.shape

    klm, lse, kls, ce = pl.pallas_call(
        _stats_kernel,
        grid=(N,),
        in_specs=[
            pl.BlockSpec((1, C, H, W), lambda n: (n, 0, 0, 0)),
            pl.BlockSpec((1, 1, H, W), lambda n: (n, 0, 0, 0)),
        ],
        out_specs=[
            pl.BlockSpec((1, 8, H, W), lambda n: (n, 0, 0, 0)),
            pl.BlockSpec((1, 1, H, W), lambda n: (n, 0, 0, 0)),
            pl.BlockSpec((1, 1, H, W), lambda n: (n, 0, 0, 0)),
            pl.BlockSpec((1, 1, H, W), lambda n: (n, 0, 0, 0)),
        ],
        out_shape=[
            jax.ShapeDtypeStruct((N, 8, H, W), jnp.bfloat16),
            jax.ShapeDtypeStruct((N, 1, H, W), jnp.float32),
            jax.ShapeDtypeStruct((N, 1, H, W), jnp.float32),
            jax.ShapeDtypeStruct((N, 1, H, W), jnp.float32),
        ],
    )(slices, targets)

    out = pl.pallas_call(
        _final_kernel,
        out_shape=jax.ShapeDtypeStruct((1, 1), jnp.float32),
        scratch_shapes=[pltpu.SMEM((256,), jnp.float32)],
    )(klm, lse, kls, ce, targets)
    return out[0, 0]
